# baseline (speedup 1.0000x reference)
import functools

import jax
import jax.numpy as jnp
from jax import lax
from jax.experimental import pallas as pl
from jax.experimental.pallas import tpu as pltpu
from jax.experimental.pallas import tpu_sc as plsc

B = 1024
N_SPECIES = 1000
N_REAC = 20000
INV_DTG = 100.0

NC = 2
NSUB = 16
NW = NC * NSUB
ROWS = B // NW
LANES = 16
GRP = 8
CHUNK = 1024
LAST = N_REAC % CHUNK
NFULL = N_REAC // CHUNK


def _sc_fused_body(rh_hbm, i0_hbm, i1_hbm, ea_hbm, lt_hbm, al_hbm, br_hbm,
                   t_hbm, g_hbm, out_hbm,
                   rh_v, t_v, g_v,
                   i0b, i1b, eab, ltb, alb, brb,
                   i0l, i1l, eal, ltl, all_, brl,
                   ob_v, ob_last_v, sem):
    wid = lax.axis_index("s") * NC + lax.axis_index("c")
    base = wid * ROWS

    pltpu.sync_copy(rh_hbm.at[pl.ds(base * N_SPECIES, ROWS * N_SPECIES)], rh_v)
    pltpu.sync_copy(t_hbm, t_v)
    pltpu.sync_copy(g_hbm, g_v)

    m_vecs = []
    s_vecs = []
    for h in range(2):
        t16 = t_v[pl.ds(base + h * LANES, LANES)]
        g16 = g_v[pl.ds(base + h * LANES, LANES)]
        m_vecs.append(-1.0 / t16)
        s_vecs.append(INV_DTG / g16)

    def run_chunk(c0, width, bufs, ob):
        i0c, i1c, eac, ltc, alc, brc = bufs
        descs = [
            pltpu.async_copy(i0_hbm.at[pl.ds(c0, width)], i0c, sem),
            pltpu.async_copy(i1_hbm.at[pl.ds(c0, width)], i1c, sem),
            pltpu.async_copy(ea_hbm.at[pl.ds(c0, width)], eac, sem),
            pltpu.async_copy(lt_hbm.at[pl.ds(c0, width)], ltc, sem),
            pltpu.async_copy(al_hbm.at[pl.ds(c0, width)], alc, sem),
            pltpu.async_copy(br_hbm.at[pl.ds(c0, width)], brc, sem),
        ]
        for d in descs:
            d.wait()

        for g in range(0, ROWS, GRP):
            m_sc = [m_vecs[(g + k) // LANES][(g + k) % LANES] for k in range(GRP)]
            s_sc = [s_vecs[(g + k) // LANES][(g + k) % LANES] for k in range(GRP)]

            def j_body(j, carry, g=g, m_sc=m_sc, s_sc=s_sc):
                o = j * LANES
                idx0 = i0c[pl.ds(o, LANES)]
                idx1 = i1c[pl.ds(o, LANES)]
                eaj = eac[pl.ds(o, LANES)]
                ltj = ltc[pl.ds(o, LANES)]
                cfj = alc[pl.ds(o, LANES)] * brc[pl.ds(o, LANES)]
                pairs = []
                for k in range(GRP):
                    bl = g + k
                    v0 = plsc.load_gather(rh_v, [idx0 + bl * N_SPECIES])
                    v1 = plsc.load_gather(rh_v, [idx1 + bl * N_SPECIES])
                    pairs.append((v0, v1))
                for k in range(GRP):
                    bl = g + k
                    rh = pairs[k][0] + pairs[k][1]
                    lp = jnp.maximum(eaj * m_sc[k], ltj)
                    ob[bl, pl.ds(o, LANES)] = (cfj * s_sc[k]) * rh * jnp.exp(lp)
                return carry

            lax.fori_loop(0, width // LANES, j_body, 0)

        pltpu.sync_copy(ob, out_hbm.at[pl.ds(base, ROWS), pl.ds(c0, width)])

    def chunk_body(c, carry):
        run_chunk(c * CHUNK, CHUNK, (i0b, i1b, eab, ltb, alb, brb), ob_v)
        return carry

    lax.fori_loop(0, NFULL, chunk_body, 0)
    run_chunk(NFULL * CHUNK, LAST, (i0l, i1l, eal, ltl, all_, brl), ob_last_v)


@functools.cache
def _sc_fused_kernel():
    return pl.kernel(
        _sc_fused_body,
        out_type=jax.ShapeDtypeStruct((B, N_REAC), jnp.float32),
        mesh=plsc.VectorSubcoreMesh(
            core_axis_name="c", subcore_axis_name="s",
            num_cores=NC, num_subcores=NSUB,
        ),
        scratch_types=[
            pltpu.VMEM((ROWS * N_SPECIES,), jnp.float32),
            pltpu.VMEM((B,), jnp.float32),
            pltpu.VMEM((B,), jnp.float32),
            pltpu.VMEM((CHUNK,), jnp.int32),
            pltpu.VMEM((CHUNK,), jnp.int32),
            pltpu.VMEM((CHUNK,), jnp.float32),
            pltpu.VMEM((CHUNK,), jnp.float32),
            pltpu.VMEM((CHUNK,), jnp.float32),
            pltpu.VMEM((CHUNK,), jnp.float32),
            pltpu.VMEM((LAST,), jnp.int32),
            pltpu.VMEM((LAST,), jnp.int32),
            pltpu.VMEM((LAST,), jnp.float32),
            pltpu.VMEM((LAST,), jnp.float32),
            pltpu.VMEM((LAST,), jnp.float32),
            pltpu.VMEM((LAST,), jnp.float32),
            pltpu.VMEM((ROWS, CHUNK), jnp.float32),
            pltpu.VMEM((ROWS, LAST), jnp.float32),
            pltpu.SemaphoreType.DMA,
        ],
        compiler_params=pltpu.CompilerParams(needs_layout_passes=False),
        name="sc_surface_reaction",
    )


@jax.jit
def kernel(rate_hopping, T_dust, den_gas, E_act, log_prob_surf_tunl, alpha,
           branching_ratio, inds_r):
    return _sc_fused_kernel()(
        rate_hopping.reshape(B * N_SPECIES),
        inds_r[:, 0], inds_r[:, 1],
        E_act, log_prob_surf_tunl, alpha, branching_ratio,
        T_dust.reshape(B), den_gas.reshape(B),
    )
